# Initial kernel scaffold; baseline (speedup 1.0000x reference)
import jax
import jax.numpy as jnp
from jax.experimental import pallas as pl


def kernel(x, w1, b1, w2, b2):
    raise NotImplementedError("write your pallas kernel here")


# speedup vs baseline: 1.3198x; 1.3198x over previous
import functools

import jax
import jax.numpy as jnp
from jax.experimental import pallas as pl
from jax.experimental.pallas import tpu as pltpu

_VMEM_BYTES = 60 * 1024 * 1024


def _se_group_kernel(x_ref, w1s_ref, b1_ref, w2_ref, b2_ref, o_ref, *, bt):
    w1s = w1s_ref[...]
    b1 = b1_ref[...]
    w2 = w2_ref[...]
    b2 = b2_ref[...]
    for b in range(bt):
        ssum = jnp.sum(x_ref[b], axis=-1, keepdims=True)
        h = jnp.sum(w1s * ssum, axis=0, keepdims=True) + b1
        h = jnp.maximum(h, 0.0)
        g = jnp.sum(w2 * h, axis=-1, keepdims=True) + b2
        g = jax.nn.sigmoid(g)
        o_ref[b] = x_ref[b] * g


def _pick_group(batch):
    for bt in (8, 4, 2):
        if batch % bt == 0:
            return bt
    return 1


def kernel(x, w1, b1, w2, b2):
    B, C, H, W = x.shape
    HW = H * W
    Cr = w1.shape[0]
    bt = _pick_group(B)

    x3 = x.reshape(B, C, HW)
    w1s = jnp.transpose(w1) * jnp.float32(1.0 / HW)

    out = pl.pallas_call(
        functools.partial(_se_group_kernel, bt=bt),
        out_shape=jax.ShapeDtypeStruct((B, C, HW), x.dtype),
        grid=(B // bt,),
        in_specs=[
            pl.BlockSpec((bt, C, HW), lambda i: (i, 0, 0)),
            pl.BlockSpec((C, Cr), lambda i: (0, 0)),
            pl.BlockSpec((1, Cr), lambda i: (0, 0)),
            pl.BlockSpec((C, Cr), lambda i: (0, 0)),
            pl.BlockSpec((C, 1), lambda i: (0, 0)),
        ],
        out_specs=pl.BlockSpec((bt, C, HW), lambda i: (i, 0, 0)),
        compiler_params=pltpu.CompilerParams(
            dimension_semantics=("parallel",),
            vmem_limit_bytes=_VMEM_BYTES,
        ),
    )(x3, w1s, b1.reshape(1, Cr), w2, b2.reshape(C, 1))
    return out.reshape(B, C, H, W)
